# baseline (speedup 1.0000x reference)
import functools

import jax
import jax.numpy as jnp
from jax import lax
from jax.experimental import pallas as pl
from jax.experimental.pallas import tpu as pltpu
from jax.experimental.pallas import tpu_sc as plsc


def _make_gather(n_b: int, n_h: int, d: int, bpc: int):
    info = plsc.get_sparse_core_info()
    nw = info.num_cores * info.num_subcores
    lanes = info.num_lanes
    assert bpc == lanes and d % lanes == 0
    assert n_b % nw == 0
    b_per_w = n_b // nw
    assert b_per_w % bpc == 0
    m = b_per_w // bpc
    assert m % 2 == 0
    chunk = bpc * n_h
    bpad = bpc + 1

    mesh = plsc.VectorSubcoreMesh(core_axis_name="c", subcore_axis_name="s")

    @functools.partial(
        pl.kernel,
        out_type=jax.ShapeDtypeStruct((n_h, d, n_b), jnp.float32),
        mesh=mesh,
        scratch_types=[
            pltpu.VMEM((2, chunk), jnp.int32),
            pltpu.VMEM((2, chunk, d), jnp.float32),
            pltpu.VMEM((2, n_h, d, bpad), jnp.float32),
        ]
        + [pltpu.SemaphoreType.DMA] * 6,
        compiler_params=pltpu.CompilerParams(
            use_tc_tiling_on_sc=False, needs_layout_passes=False
        ),
    )
    def gather_kernel(idx_hbm, table_hbm, out_hbm, idx_v, rows_v, tp_v, *sems):
        gsem = sems[:2]
        ssem = sems[2:4]
        isem = sems[4:]
        wid = lax.axis_index("s") * info.num_cores + lax.axis_index("c")
        base_b = wid * b_per_w

        def idx_copy(ci, slot):
            off = pl.multiple_of((base_b + ci * bpc) * n_h, 8)
            return pltpu.make_async_copy(
                idx_hbm.at[pl.ds(off, chunk)], idx_v.at[slot], isem[slot]
            )

        def gather_copy(slot):
            return pltpu.make_async_copy(
                table_hbm.at[idx_v.at[slot]], rows_v.at[slot], gsem[slot]
            )

        def store_copy(ci, slot):
            boff = pl.multiple_of(base_b + ci * bpc, 8)
            return pltpu.make_async_copy(
                tp_v.at[slot, :, :, pl.ds(0, bpc)],
                out_hbm.at[:, :, pl.ds(boff, bpc)],
                ssem[slot],
            )

        lane = lax.iota(jnp.int32, lanes)

        def transpose_chunk(slot):
            src = rows_v.at[slot]
            dst = tp_v.at[slot]

            unroll = 5
            assert n_h % unroll == 0
            cvs = [lane + k * lanes for k in range(d // lanes)]

            def b_body(b, carry):
                bv = jnp.full((lanes,), b, jnp.int32)
                r0 = b * n_h

                def h_body(h0, carry2):
                    for u in range(unroll):
                        h = h0 + u
                        hv = jnp.full((lanes,), h, jnp.int32)
                        for k in range(d // lanes):
                            vec = src[r0 + h, pl.ds(k * lanes, lanes)]
                            plsc.store_scatter(dst, [hv, cvs[k], bv], vec)
                    return carry2

                lax.fori_loop(0, n_h // unroll, lambda t, c: h_body(t * unroll, c), 0)
                return carry

            lax.fori_loop(0, bpc, b_body, 0)

        idx_copy(0, 0).start()
        idx_copy(1, 1).start()
        idx_copy(0, 0).wait()
        gather_copy(0).start()
        idx_copy(1, 1).wait()
        gather_copy(1).start()

        def outer_body(o, carry):
            for slot in range(2):
                i = o * 2 + slot
                gather_copy(slot).wait()

                @pl.when(i + 2 < m)
                def _():
                    idx_copy(i + 2, slot).start()

                @pl.when(o > 0)
                def _():
                    store_copy(i - 2, slot).wait()

                transpose_chunk(slot)

                @pl.when(i + 2 < m)
                def _():
                    idx_copy(i + 2, slot).wait()
                    gather_copy(slot).start()

                store_copy(i, slot).start()
            return carry

        lax.fori_loop(0, m // 2, outer_body, 0)
        store_copy(m - 2, 0).wait()
        store_copy(m - 1, 1).wait()

    return gather_kernel


def kernel(in_feat, table):
    b, h = in_feat.shape
    v, d = table.shape
    idx = in_feat.reshape(b * h).astype(jnp.int32)
    out = _make_gather(b, h, d, bpc=16)(idx, table)
    return jnp.transpose(out, (2, 0, 1))


# speedup vs baseline: 1.0149x; 1.0149x over previous
import functools

import jax
import jax.numpy as jnp
from jax import lax
from jax.experimental import pallas as pl
from jax.experimental.pallas import tpu as pltpu
from jax.experimental.pallas import tpu_sc as plsc


def _make_gather(n_b: int, n_h: int, d: int, bpc: int):
    info = plsc.get_sparse_core_info()
    nw = info.num_cores * info.num_subcores
    lanes = info.num_lanes
    assert bpc == lanes and d % lanes == 0
    assert n_b % nw == 0
    b_per_w = n_b // nw
    assert b_per_w % bpc == 0
    m = b_per_w // bpc
    assert m % 2 == 0
    chunk = bpc * n_h
    bpad = bpc + 1

    mesh = plsc.VectorSubcoreMesh(core_axis_name="c", subcore_axis_name="s")

    @functools.partial(
        pl.kernel,
        out_type=jax.ShapeDtypeStruct((n_h, d, n_b), jnp.float32),
        mesh=mesh,
        scratch_types=[
            pltpu.VMEM((2, chunk), jnp.int32),
            pltpu.VMEM((2, chunk, d), jnp.float32),
            pltpu.VMEM((2, n_h, d, bpad), jnp.float32),
        ]
        + [pltpu.SemaphoreType.DMA] * 6,
        compiler_params=pltpu.CompilerParams(
            use_tc_tiling_on_sc=False, needs_layout_passes=False
        ),
    )
    def gather_kernel(idx_hbm, table_hbm, out_hbm, idx_v, rows_v, tp_v, *sems):
        gsem = sems[:2]
        ssem = sems[2:4]
        isem = sems[4:]
        wid = lax.axis_index("s") * info.num_cores + lax.axis_index("c")
        base_b = wid * b_per_w

        def idx_copy(ci, slot):
            off = pl.multiple_of((base_b + ci * bpc) * n_h, 8)
            return pltpu.make_async_copy(
                idx_hbm.at[pl.ds(off, chunk)], idx_v.at[slot], isem[slot]
            )

        def gather_copy(slot):
            return pltpu.make_async_copy(
                table_hbm.at[idx_v.at[slot]], rows_v.at[slot], gsem[slot]
            )

        def store_copy(ci, slot):
            boff = pl.multiple_of(base_b + ci * bpc, 8)
            return pltpu.make_async_copy(
                tp_v.at[slot, :, :, pl.ds(0, bpc)],
                out_hbm.at[:, :, pl.ds(boff, bpc)],
                ssem[slot],
            )

        lane = lax.iota(jnp.int32, lanes)

        def transpose_chunk(slot):
            src = rows_v.at[slot]
            dst = tp_v.at[slot]

            def r_body(r, carry):
                b = r // n_h
                h = r - b * n_h
                hv = jnp.full((lanes,), h, jnp.int32)
                bv = jnp.full((lanes,), b, jnp.int32)
                for k in range(d // lanes):
                    vec = src[r, pl.ds(k * lanes, lanes)]
                    plsc.store_scatter(dst, [hv, lane + k * lanes, bv], vec)
                return carry

            lax.fori_loop(0, chunk, r_body, 0)

        idx_copy(0, 0).start()
        idx_copy(1, 1).start()
        idx_copy(0, 0).wait()
        gather_copy(0).start()
        idx_copy(1, 1).wait()
        gather_copy(1).start()

        def outer_body(o, carry):
            for slot in range(2):
                i = o * 2 + slot
                gather_copy(slot).wait()

                @pl.when(i + 2 < m)
                def _():
                    idx_copy(i + 2, slot).start()

                @pl.when(o > 0)
                def _():
                    store_copy(i - 2, slot).wait()

                transpose_chunk(slot)

                @pl.when(i + 2 < m)
                def _():
                    idx_copy(i + 2, slot).wait()
                    gather_copy(slot).start()

                store_copy(i, slot).start()
            return carry

        lax.fori_loop(0, m // 2, outer_body, 0)
        store_copy(m - 2, 0).wait()
        store_copy(m - 1, 1).wait()

    return gather_kernel


def kernel(in_feat, table):
    b, h = in_feat.shape
    v, d = table.shape
    idx = in_feat.reshape(b * h).astype(jnp.int32)
    out = _make_gather(b, h, d, bpc=16)(idx, table)
    return jnp.transpose(out, (2, 0, 1))
